# baseline (speedup 1.0000x reference)
import functools

import jax
import jax.numpy as jnp
from jax import lax
from jax.experimental import pallas as pl
from jax.experimental.pallas import tpu as pltpu
from jax.experimental.pallas import tpu_sc as plsc

_N = 10000
_E = 320000
_IN_C = 128
_HID_C = 128
_OUT_C = 64
_K_HOPS = 5
_ALPHA = 0.1

_NP = 10240
_TILES = 32
_CH = 128
_ET = _E + _N
_NB = 6
_LOOK = 3
_J = 84
_EPAD = _TILES * _J * _CH
_SLICE = _NP // 16

_mesh = plsc.VectorSubcoreMesh(core_axis_name="c", subcore_axis_name="s")
_sc_params = pltpu.CompilerParams(use_tc_tiling_on_sc=False)


@functools.partial(
    pl.kernel,
    out_type=jax.ShapeDtypeStruct((2, _NP, 16), jnp.float32),
    mesh=_mesh,
    scratch_types=[
        pltpu.VMEM_SHARED((_NP, 16), jnp.float32),
        pltpu.VMEM((_J, _CH), jnp.int32),
        pltpu.VMEM((_CH, 16), jnp.float32),
    ],
    compiler_params=_sc_params,
)
def _deg_kernel(col_hbm, zer_hbm, ones_hbm, part_hbm, acc, col_v, ones_v):
    c = lax.axis_index("c")
    s = lax.axis_index("s")
    w = c * 16 + s
    pltpu.sync_copy(zer_hbm.at[0, pl.ds(s * _SLICE, _SLICE)],
                    acc.at[pl.ds(s * _SLICE, _SLICE)])
    pltpu.sync_copy(col_hbm.at[w], col_v)
    pltpu.sync_copy(ones_hbm, ones_v)
    plsc.subcore_barrier()

    @pl.loop(0, _J)
    def _(j):
        pltpu.sync_copy(ones_v, acc.at[col_v.at[j]], add=True)

    plsc.subcore_barrier()
    pltpu.sync_copy(acc.at[pl.ds(s * _SLICE, _SLICE)],
                    part_hbm.at[c, pl.ds(s * _SLICE, _SLICE)])


@functools.partial(
    pl.kernel,
    out_type=jax.ShapeDtypeStruct((2, _NP, _OUT_C), jnp.float32),
    mesh=_mesh,
    scratch_types=[
        pltpu.VMEM_SHARED((_NP, _OUT_C), jnp.float32),
        pltpu.VMEM((_J, _CH), jnp.int32),
        pltpu.VMEM((_J, _CH), jnp.int32),
    ]
    + [pltpu.VMEM((_CH, _OUT_C), jnp.float32)] * _NB
    + [pltpu.SemaphoreType.DMA] * (2 * _NB),
    compiler_params=_sc_params,
)
def _hop_kernel(u_hbm, row_hbm, col_hbm, zer_hbm, part_hbm,
                acc, row_v, col_v, *bufs_and_sems):
    bufs = bufs_and_sems[:_NB]
    gsem = bufs_and_sems[_NB:2 * _NB]
    ssem = bufs_and_sems[2 * _NB:]
    c = lax.axis_index("c")
    s = lax.axis_index("s")
    w = c * 16 + s
    pltpu.sync_copy(zer_hbm.at[pl.ds(s * _SLICE, _SLICE)],
                    acc.at[pl.ds(s * _SLICE, _SLICE)])
    pltpu.sync_copy(row_hbm.at[w], row_v)
    pltpu.sync_copy(col_hbm.at[w], col_v)
    plsc.subcore_barrier()

    for k in range(_LOOK):
        pltpu.async_copy(u_hbm.at[row_v.at[k]], bufs[k], gsem[k])

    @pl.loop(0, _J, step=_NB)
    def _(j0):
        for k in range(_NB):
            j = j0 + k
            bg = (k + _LOOK) % _NB

            @pl.when(j + _LOOK < _J)
            def _():
                @pl.when(j + _LOOK >= _NB)
                def _():
                    pltpu.make_async_copy(
                        bufs[bg], acc.at[col_v.at[j]], ssem[bg]).wait()
                pltpu.async_copy(u_hbm.at[row_v.at[j + _LOOK]],
                                 bufs[bg], gsem[bg])

            pltpu.make_async_copy(u_hbm.at[row_v.at[j]], bufs[k],
                                  gsem[k]).wait()
            pltpu.async_copy(bufs[k], acc.at[col_v.at[j]], ssem[k],
                             add=True)

    for k in range(_NB):
        pltpu.make_async_copy(bufs[k], acc.at[col_v.at[0]], ssem[k]).wait()

    plsc.subcore_barrier()
    pltpu.sync_copy(acc.at[pl.ds(s * _SLICE, _SLICE)],
                    part_hbm.at[c, pl.ds(s * _SLICE, _SLICE)])


def _prep_body(x_ref, w1_ref, b1_ref, w2_ref, b2_ref, d0_ref, d1_ref,
               h_ref, u0_ref, dis_ref):
    h1 = jnp.maximum(x_ref[...] @ w1_ref[...] + b1_ref[...], 0.0)
    h = h1 @ w2_ref[...] + b2_ref[...]
    deg = d0_ref[:, 0:1] + d1_ref[:, 0:1]
    dis = jnp.where(deg > 0.0, lax.rsqrt(deg), 0.0)
    h_ref[...] = h
    u0_ref[...] = dis * h
    dis_ref[...] = jnp.broadcast_to(dis, h.shape)


def _prep(xp, W1, b1, W2, b2, d0, d1):
    B = 1024
    g = _NP // B
    fullspec = lambda shape: pl.BlockSpec(shape, lambda i: (0, 0))
    rowspec = lambda shape: pl.BlockSpec(shape, lambda i: (i, 0))
    return pl.pallas_call(
        _prep_body,
        grid=(g,),
        in_specs=[rowspec((B, _IN_C)), fullspec((_IN_C, _HID_C)),
                  fullspec((1, _HID_C)), fullspec((_HID_C, _OUT_C)),
                  fullspec((1, _OUT_C)), rowspec((B, 16)), rowspec((B, 16))],
        out_specs=[rowspec((B, _OUT_C))] * 3,
        out_shape=[jax.ShapeDtypeStruct((_NP, _OUT_C), jnp.float32)] * 3,
    )(xp, W1, b1, W2, b2, d0, d1)


def _comb_body(p0_ref, p1_ref, dis_ref, u0_ref, out_ref):
    dis = dis_ref[...]
    agg = p0_ref[...] + p1_ref[...]
    out_ref[...] = (1.0 - _ALPHA) * dis * dis * agg + _ALPHA * u0_ref[...]


def _comb(p0, p1, dis, u0):
    B = 1024
    g = _NP // B
    spec = pl.BlockSpec((B, _OUT_C), lambda i: (i, 0))
    return pl.pallas_call(
        _comb_body,
        grid=(g,),
        in_specs=[spec] * 4,
        out_specs=spec,
        out_shape=jax.ShapeDtypeStruct((_NP, _OUT_C), jnp.float32),
    )(p0, p1, dis, u0)


def _fin_body(p0_ref, p1_ref, dis_ref, h_ref, out_ref):
    t = ((1.0 - _ALPHA) * dis_ref[...] * (p0_ref[...] + p1_ref[...])
         + _ALPHA * h_ref[...])
    m = jnp.max(t, axis=1, keepdims=True)
    e = jnp.exp(t - m)
    out_ref[...] = (t - m) - jnp.log(jnp.sum(e, axis=1, keepdims=True))


def _fin(p0, p1, dis, h):
    B = 1000
    g = _N // B
    spec = pl.BlockSpec((B, _OUT_C), lambda i: (i, 0))
    return pl.pallas_call(
        _fin_body,
        grid=(g,),
        in_specs=[spec] * 4,
        out_specs=spec,
        out_shape=jax.ShapeDtypeStruct((_N, _OUT_C), jnp.float32),
    )(p0, p1, dis, h)


def kernel(x, edge_index, W1, b1, W2, b2):
    row = edge_index[0].astype(jnp.int32)
    col = edge_index[1].astype(jnp.int32)
    loop = jnp.arange(_N, dtype=jnp.int32)
    pad = _EPAD - _ET
    row = jnp.concatenate([row, loop, jnp.zeros((pad,), jnp.int32)])
    col = jnp.concatenate(
        [col, loop, _N + (jnp.arange(pad, dtype=jnp.int32) % (_NP - _N))])
    rowt = row.reshape(_J, _TILES, _CH).transpose(1, 0, 2)
    colt = col.reshape(_J, _TILES, _CH).transpose(1, 0, 2)

    zer = jnp.zeros((_NP, _OUT_C), jnp.float32)
    zer16 = jnp.zeros((1, _NP, 16), jnp.float32)
    ones16 = jnp.ones((_CH, 16), jnp.float32)

    dpart = _deg_kernel(colt, zer16, ones16)

    xp = jnp.concatenate([x, jnp.zeros((_NP - _N, _IN_C), jnp.float32)])
    h, u0, dis = _prep(xp, W1, b1.reshape(1, -1), W2, b2.reshape(1, -1),
                       dpart[0], dpart[1])

    u = u0
    for _ in range(_K_HOPS - 1):
        p = _hop_kernel(u, rowt, colt, zer)
        u = _comb(p[0], p[1], dis, u0)
    p = _hop_kernel(u, rowt, colt, zer)
    return _fin(p[0][:_N], p[1][:_N], dis[:_N], h[:_N])


# speedup vs baseline: 2.2943x; 2.2943x over previous
import functools

import jax
import jax.numpy as jnp
from jax import lax
from jax.experimental import pallas as pl
from jax.experimental.pallas import tpu as pltpu
from jax.experimental.pallas import tpu_sc as plsc

_N = 10000
_E = 320000
_IN_C = 128
_HID_C = 128
_OUT_C = 64
_K_HOPS = 5
_ALPHA = 0.1

_NP = 10240
_TILES = 32
_CH = 128
_ET = _E + _N
_NB = 6
_LOOK = 3
_J = 82
_EPAD = _TILES * _J * _CH
_SLICE = _NP // 16

_mesh = plsc.VectorSubcoreMesh(core_axis_name="c", subcore_axis_name="s")
_sc_params = pltpu.CompilerParams(use_tc_tiling_on_sc=False)


@functools.partial(
    pl.kernel,
    out_type=jax.ShapeDtypeStruct((2, _NP, 16), jnp.float32),
    mesh=_mesh,
    scratch_types=[
        pltpu.VMEM_SHARED((_NP, 16), jnp.float32),
        pltpu.VMEM((_J, _CH), jnp.int32),
        pltpu.VMEM((_CH, 16), jnp.float32),
    ],
    compiler_params=_sc_params,
)
def _deg_kernel(col_hbm, zer_hbm, ones_hbm, part_hbm, acc, col_v, ones_v):
    c = lax.axis_index("c")
    s = lax.axis_index("s")
    w = c * 16 + s
    pltpu.sync_copy(zer_hbm.at[0, pl.ds(s * _SLICE, _SLICE)],
                    acc.at[pl.ds(s * _SLICE, _SLICE)])
    pltpu.sync_copy(col_hbm.at[w], col_v)
    pltpu.sync_copy(ones_hbm, ones_v)
    plsc.subcore_barrier()

    @pl.loop(0, _J)
    def _(j):
        pltpu.sync_copy(ones_v, acc.at[col_v.at[j]], add=True)

    plsc.subcore_barrier()
    pltpu.sync_copy(acc.at[pl.ds(s * _SLICE, _SLICE)],
                    part_hbm.at[c, pl.ds(s * _SLICE, _SLICE)])


@functools.partial(
    pl.kernel,
    out_type=jax.ShapeDtypeStruct((2, _NP, _OUT_C), jnp.float32),
    mesh=_mesh,
    scratch_types=[
        pltpu.VMEM_SHARED((_NP, _OUT_C), jnp.float32),
        pltpu.VMEM_SHARED((_NP, _OUT_C), jnp.float32),
        pltpu.VMEM((_J, _CH), jnp.int32),
        pltpu.VMEM((_J, _CH), jnp.int32),
    ]
    + [pltpu.VMEM((_CH, _OUT_C), jnp.float32)] * _NB
    + [pltpu.SemaphoreType.DMA] * (2 * _NB),
    compiler_params=_sc_params,
)
def _hop_kernel(u_hbm, row_hbm, col_hbm, zer_hbm, part_hbm,
                acc, u_sp, row_v, col_v, *bufs_and_sems):
    bufs = bufs_and_sems[:_NB]
    gsem = bufs_and_sems[_NB:2 * _NB]
    c = lax.axis_index("c")
    s = lax.axis_index("s")
    w = c * 16 + s
    pltpu.sync_copy(zer_hbm.at[pl.ds(s * _SLICE, _SLICE)],
                    acc.at[pl.ds(s * _SLICE, _SLICE)])
    pltpu.sync_copy(u_hbm.at[pl.ds(s * _SLICE, _SLICE)],
                    u_sp.at[pl.ds(s * _SLICE, _SLICE)])
    pltpu.sync_copy(row_hbm.at[w], row_v)
    pltpu.sync_copy(col_hbm.at[w], col_v)
    plsc.subcore_barrier()

    @pl.loop(0, _J, step=2)
    def _(j):
        cp0 = pltpu.async_copy(u_sp.at[row_v.at[j]], bufs[0], gsem[0])
        cp1 = pltpu.async_copy(u_sp.at[row_v.at[j + 1]], bufs[1], gsem[1])
        cp0.wait()
        pltpu.sync_copy(bufs[0], acc.at[col_v.at[j]], add=True)
        cp1.wait()
        pltpu.sync_copy(bufs[1], acc.at[col_v.at[j + 1]], add=True)

    plsc.subcore_barrier()
    pltpu.sync_copy(acc.at[pl.ds(s * _SLICE, _SLICE)],
                    part_hbm.at[c, pl.ds(s * _SLICE, _SLICE)])


def _prep_body(x_ref, w1_ref, b1_ref, w2_ref, b2_ref, d0_ref, d1_ref,
               h_ref, u0_ref, dis_ref):
    h1 = jnp.maximum(x_ref[...] @ w1_ref[...] + b1_ref[...], 0.0)
    h = h1 @ w2_ref[...] + b2_ref[...]
    deg = d0_ref[:, 0:1] + d1_ref[:, 0:1]
    dis = jnp.where(deg > 0.0, lax.rsqrt(deg), 0.0)
    h_ref[...] = h
    u0_ref[...] = dis * h
    dis_ref[...] = jnp.broadcast_to(dis, h.shape)


def _prep(xp, W1, b1, W2, b2, d0, d1):
    B = 1024
    g = _NP // B
    fullspec = lambda shape: pl.BlockSpec(shape, lambda i: (0, 0))
    rowspec = lambda shape: pl.BlockSpec(shape, lambda i: (i, 0))
    return pl.pallas_call(
        _prep_body,
        grid=(g,),
        in_specs=[rowspec((B, _IN_C)), fullspec((_IN_C, _HID_C)),
                  fullspec((1, _HID_C)), fullspec((_HID_C, _OUT_C)),
                  fullspec((1, _OUT_C)), rowspec((B, 16)), rowspec((B, 16))],
        out_specs=[rowspec((B, _OUT_C))] * 3,
        out_shape=[jax.ShapeDtypeStruct((_NP, _OUT_C), jnp.float32)] * 3,
    )(xp, W1, b1, W2, b2, d0, d1)


def _comb_body(p0_ref, p1_ref, dis_ref, u0_ref, out_ref):
    dis = dis_ref[...]
    agg = p0_ref[...] + p1_ref[...]
    out_ref[...] = (1.0 - _ALPHA) * dis * dis * agg + _ALPHA * u0_ref[...]


def _comb(p0, p1, dis, u0):
    B = 1024
    g = _NP // B
    spec = pl.BlockSpec((B, _OUT_C), lambda i: (i, 0))
    return pl.pallas_call(
        _comb_body,
        grid=(g,),
        in_specs=[spec] * 4,
        out_specs=spec,
        out_shape=jax.ShapeDtypeStruct((_NP, _OUT_C), jnp.float32),
    )(p0, p1, dis, u0)


def _fin_body(p0_ref, p1_ref, dis_ref, h_ref, out_ref):
    t = ((1.0 - _ALPHA) * dis_ref[...] * (p0_ref[...] + p1_ref[...])
         + _ALPHA * h_ref[...])
    m = jnp.max(t, axis=1, keepdims=True)
    e = jnp.exp(t - m)
    out_ref[...] = (t - m) - jnp.log(jnp.sum(e, axis=1, keepdims=True))


def _fin(p0, p1, dis, h):
    B = 1000
    g = _N // B
    spec = pl.BlockSpec((B, _OUT_C), lambda i: (i, 0))
    return pl.pallas_call(
        _fin_body,
        grid=(g,),
        in_specs=[spec] * 4,
        out_specs=spec,
        out_shape=jax.ShapeDtypeStruct((_N, _OUT_C), jnp.float32),
    )(p0, p1, dis, h)


def kernel(x, edge_index, W1, b1, W2, b2):
    row = edge_index[0].astype(jnp.int32)
    col = edge_index[1].astype(jnp.int32)
    loop = jnp.arange(_N, dtype=jnp.int32)
    pad = _EPAD - _ET
    row = jnp.concatenate([row, loop, jnp.zeros((pad,), jnp.int32)])
    col = jnp.concatenate(
        [col, loop, _N + (jnp.arange(pad, dtype=jnp.int32) % (_NP - _N))])
    rowt = row.reshape(_J, _TILES, _CH).transpose(1, 0, 2)
    colt = col.reshape(_J, _TILES, _CH).transpose(1, 0, 2)

    zer = jnp.zeros((_NP, _OUT_C), jnp.float32)
    zer16 = jnp.zeros((1, _NP, 16), jnp.float32)
    ones16 = jnp.ones((_CH, 16), jnp.float32)

    dpart = _deg_kernel(colt, zer16, ones16)

    xp = jnp.concatenate([x, jnp.zeros((_NP - _N, _IN_C), jnp.float32)])
    h, u0, dis = _prep(xp, W1, b1.reshape(1, -1), W2, b2.reshape(1, -1),
                       dpart[0], dpart[1])

    u = u0
    for _ in range(_K_HOPS - 1):
        p = _hop_kernel(u, rowt, colt, zer)
        u = _comb(p[0], p[1], dis, u0)
    p = _hop_kernel(u, rowt, colt, zer)
    return _fin(p[0][:_N], p[1][:_N], dis[:_N], h[:_N])


# speedup vs baseline: 2.8335x; 1.2350x over previous
import functools

import jax
import jax.numpy as jnp
from jax import lax
from jax.experimental import pallas as pl
from jax.experimental.pallas import tpu as pltpu
from jax.experimental.pallas import tpu_sc as plsc

_N = 10000
_E = 320000
_IN_C = 128
_HID_C = 128
_OUT_C = 64
_K_HOPS = 5
_ALPHA = 0.1

_NP = 10240
_TILES = 32
_CH = 128
_ET = _E + _N
_NB = 4
_J = 84
_SEG = 28
_EPAD = _TILES * _J * _CH
_SLICE = _NP // 16

_mesh = plsc.VectorSubcoreMesh(core_axis_name="c", subcore_axis_name="s")
_sc_params = pltpu.CompilerParams(use_tc_tiling_on_sc=False)


@functools.partial(
    pl.kernel,
    out_type=jax.ShapeDtypeStruct((2, _NP, 16), jnp.float32),
    mesh=_mesh,
    scratch_types=[
        pltpu.VMEM_SHARED((_NP, 16), jnp.float32),
        pltpu.VMEM((_J, _CH), jnp.int32),
        pltpu.VMEM((_CH, 16), jnp.float32),
    ],
    compiler_params=_sc_params,
)
def _deg_kernel(col_hbm, zer_hbm, ones_hbm, part_hbm, acc, col_v, ones_v):
    c = lax.axis_index("c")
    s = lax.axis_index("s")
    w = c * 16 + s
    pltpu.sync_copy(zer_hbm.at[0, pl.ds(s * _SLICE, _SLICE)],
                    acc.at[pl.ds(s * _SLICE, _SLICE)])
    pltpu.sync_copy(col_hbm.at[w], col_v)
    pltpu.sync_copy(ones_hbm, ones_v)
    plsc.subcore_barrier()

    @pl.loop(0, _J)
    def _(j):
        pltpu.sync_copy(ones_v, acc.at[col_v.at[j]], add=True)

    plsc.subcore_barrier()
    pltpu.sync_copy(acc.at[pl.ds(s * _SLICE, _SLICE)],
                    part_hbm.at[c, pl.ds(s * _SLICE, _SLICE)])


@functools.partial(
    pl.kernel,
    out_type=jax.ShapeDtypeStruct((2, _NP, _OUT_C), jnp.float32),
    mesh=_mesh,
    scratch_types=[
        pltpu.VMEM_SHARED((_NP, _OUT_C), jnp.float32),
        pltpu.VMEM_SHARED((_NP, _OUT_C), jnp.float32),
        pltpu.VMEM((_SEG, _CH), jnp.int32),
        pltpu.VMEM((_SEG, _CH), jnp.int32),
    ]
    + [pltpu.VMEM((_CH, _OUT_C), jnp.float32)] * _NB
    + [pltpu.SemaphoreType.DMA] * (2 * _NB),
    compiler_params=_sc_params,
)
def _hop_kernel(u_hbm, row_hbm, col_hbm, zer_hbm, part_hbm,
                acc, u_sp, row_v, col_v, *bufs_and_sems):
    bufs = bufs_and_sems[:_NB]
    gsem = bufs_and_sems[_NB:2 * _NB]
    c = lax.axis_index("c")
    s = lax.axis_index("s")
    w = c * 16 + s
    pltpu.sync_copy(zer_hbm.at[pl.ds(s * _SLICE, _SLICE)],
                    acc.at[pl.ds(s * _SLICE, _SLICE)])
    pltpu.sync_copy(u_hbm.at[pl.ds(s * _SLICE, _SLICE)],
                    u_sp.at[pl.ds(s * _SLICE, _SLICE)])
    plsc.subcore_barrier()

    @pl.loop(0, _J // _SEG)
    def _(g):
        pltpu.sync_copy(row_hbm.at[w, pl.ds(g * _SEG, _SEG)], row_v)
        pltpu.sync_copy(col_hbm.at[w, pl.ds(g * _SEG, _SEG)], col_v)

        @pl.loop(0, _SEG, step=_NB)
        def _(j):
            cps = [pltpu.async_copy(u_sp.at[row_v.at[j + b]],
                                    bufs[b], gsem[b])
                   for b in range(_NB)]
            for b in range(_NB):
                cps[b].wait()
                pltpu.sync_copy(bufs[b], acc.at[col_v.at[j + b]], add=True)

    plsc.subcore_barrier()
    pltpu.sync_copy(acc.at[pl.ds(s * _SLICE, _SLICE)],
                    part_hbm.at[c, pl.ds(s * _SLICE, _SLICE)])


def _prep_body(x_ref, w1_ref, b1_ref, w2_ref, b2_ref, d0_ref, d1_ref,
               h_ref, u0_ref, dis_ref):
    h1 = jnp.maximum(x_ref[...] @ w1_ref[...] + b1_ref[...], 0.0)
    h = h1 @ w2_ref[...] + b2_ref[...]
    deg = d0_ref[:, 0:1] + d1_ref[:, 0:1]
    dis = jnp.where(deg > 0.0, lax.rsqrt(deg), 0.0)
    h_ref[...] = h
    u0_ref[...] = dis * h
    dis_ref[...] = jnp.broadcast_to(dis, h.shape)


def _prep(xp, W1, b1, W2, b2, d0, d1):
    B = 1024
    g = _NP // B
    fullspec = lambda shape: pl.BlockSpec(shape, lambda i: (0, 0))
    rowspec = lambda shape: pl.BlockSpec(shape, lambda i: (i, 0))
    return pl.pallas_call(
        _prep_body,
        grid=(g,),
        in_specs=[rowspec((B, _IN_C)), fullspec((_IN_C, _HID_C)),
                  fullspec((1, _HID_C)), fullspec((_HID_C, _OUT_C)),
                  fullspec((1, _OUT_C)), rowspec((B, 16)), rowspec((B, 16))],
        out_specs=[rowspec((B, _OUT_C))] * 3,
        out_shape=[jax.ShapeDtypeStruct((_NP, _OUT_C), jnp.float32)] * 3,
    )(xp, W1, b1, W2, b2, d0, d1)


def _comb_body(p0_ref, p1_ref, dis_ref, u0_ref, out_ref):
    dis = dis_ref[...]
    agg = p0_ref[...] + p1_ref[...]
    out_ref[...] = (1.0 - _ALPHA) * dis * dis * agg + _ALPHA * u0_ref[...]


def _comb(p0, p1, dis, u0):
    B = 1024
    g = _NP // B
    spec = pl.BlockSpec((B, _OUT_C), lambda i: (i, 0))
    return pl.pallas_call(
        _comb_body,
        grid=(g,),
        in_specs=[spec] * 4,
        out_specs=spec,
        out_shape=jax.ShapeDtypeStruct((_NP, _OUT_C), jnp.float32),
    )(p0, p1, dis, u0)


def _fin_body(p0_ref, p1_ref, dis_ref, h_ref, out_ref):
    t = ((1.0 - _ALPHA) * dis_ref[...] * (p0_ref[...] + p1_ref[...])
         + _ALPHA * h_ref[...])
    m = jnp.max(t, axis=1, keepdims=True)
    e = jnp.exp(t - m)
    out_ref[...] = (t - m) - jnp.log(jnp.sum(e, axis=1, keepdims=True))


def _fin(p0, p1, dis, h):
    B = 1000
    g = _N // B
    spec = pl.BlockSpec((B, _OUT_C), lambda i: (i, 0))
    return pl.pallas_call(
        _fin_body,
        grid=(g,),
        in_specs=[spec] * 4,
        out_specs=spec,
        out_shape=jax.ShapeDtypeStruct((_N, _OUT_C), jnp.float32),
    )(p0, p1, dis, h)


def kernel(x, edge_index, W1, b1, W2, b2):
    row = edge_index[0].astype(jnp.int32)
    col = edge_index[1].astype(jnp.int32)
    loop = jnp.arange(_N, dtype=jnp.int32)
    pad = _EPAD - _ET
    row = jnp.concatenate([row, loop, jnp.zeros((pad,), jnp.int32)])
    col = jnp.concatenate(
        [col, loop, _N + (jnp.arange(pad, dtype=jnp.int32) % (_NP - _N))])
    rowt = row.reshape(_J, _TILES, _CH).transpose(1, 0, 2)
    colt = col.reshape(_J, _TILES, _CH).transpose(1, 0, 2)

    zer = jnp.zeros((_NP, _OUT_C), jnp.float32)
    zer16 = jnp.zeros((1, _NP, 16), jnp.float32)
    ones16 = jnp.ones((_CH, 16), jnp.float32)

    dpart = _deg_kernel(colt, zer16, ones16)

    xp = jnp.concatenate([x, jnp.zeros((_NP - _N, _IN_C), jnp.float32)])
    h, u0, dis = _prep(xp, W1, b1.reshape(1, -1), W2, b2.reshape(1, -1),
                       dpart[0], dpart[1])

    u = u0
    for _ in range(_K_HOPS - 1):
        p = _hop_kernel(u, rowt, colt, zer)
        u = _comb(p[0], p[1], dis, u0)
    p = _hop_kernel(u, rowt, colt, zer)
    return _fin(p[0][:_N], p[1][:_N], dis[:_N], h[:_N])


# speedup vs baseline: 2.8652x; 1.0112x over previous
import functools

import jax
import jax.numpy as jnp
from jax import lax
from jax.experimental import pallas as pl
from jax.experimental.pallas import tpu as pltpu
from jax.experimental.pallas import tpu_sc as plsc

_N = 10000
_E = 320000
_IN_C = 128
_HID_C = 128
_OUT_C = 64
_K_HOPS = 5
_ALPHA = 0.1

_NP = 10240
_TILES = 32
_CH = 128
_ET = _E + _N
_NB = 4
_J = 84
_SEG = 28
_EPAD = _TILES * _J * _CH
_SLICE = _NP // 16

_mesh = plsc.VectorSubcoreMesh(core_axis_name="c", subcore_axis_name="s")
_sc_params = pltpu.CompilerParams(use_tc_tiling_on_sc=False)


@functools.partial(
    pl.kernel,
    out_type=jax.ShapeDtypeStruct((2, _NP, 16), jnp.float32),
    mesh=_mesh,
    scratch_types=[
        pltpu.VMEM_SHARED((_NP, 16), jnp.float32),
        pltpu.VMEM((_J, _CH), jnp.int32),
        pltpu.VMEM((_CH, 16), jnp.float32),
    ],
    compiler_params=_sc_params,
)
def _deg_kernel(col_hbm, zer_hbm, ones_hbm, part_hbm, acc, col_v, ones_v):
    c = lax.axis_index("c")
    s = lax.axis_index("s")
    w = c * 16 + s
    pltpu.sync_copy(zer_hbm.at[0, pl.ds(s * _SLICE, _SLICE)],
                    acc.at[pl.ds(s * _SLICE, _SLICE)])
    pltpu.sync_copy(col_hbm.at[w], col_v)
    pltpu.sync_copy(ones_hbm, ones_v)
    plsc.subcore_barrier()

    @pl.loop(0, _J)
    def _(j):
        pltpu.sync_copy(ones_v, acc.at[col_v.at[j]], add=True)

    plsc.subcore_barrier()
    pltpu.sync_copy(acc.at[pl.ds(s * _SLICE, _SLICE)],
                    part_hbm.at[c, pl.ds(s * _SLICE, _SLICE)])


@functools.partial(
    pl.kernel,
    out_type=jax.ShapeDtypeStruct((2, _NP, _OUT_C), jnp.float32),
    mesh=_mesh,
    scratch_types=[
        pltpu.VMEM_SHARED((_NP, _OUT_C), jnp.float32),
        pltpu.VMEM_SHARED((_NP, _OUT_C), jnp.float32),
        pltpu.VMEM((_SEG, _CH), jnp.int32),
        pltpu.VMEM((_SEG, _CH), jnp.int32),
    ]
    + [pltpu.VMEM((_CH, _OUT_C), jnp.float32)] * _NB
    + [pltpu.SemaphoreType.DMA] * (2 * _NB),
    compiler_params=_sc_params,
)
def _hop_kernel(u_hbm, row_hbm, col_hbm, zer_hbm, part_hbm,
                acc, u_sp, row_v, col_v, *bufs_and_sems):
    bufs = bufs_and_sems[:_NB]
    gsem = bufs_and_sems[_NB:2 * _NB]
    c = lax.axis_index("c")
    s = lax.axis_index("s")
    w = c * 16 + s
    pltpu.sync_copy(zer_hbm.at[pl.ds(s * _SLICE, _SLICE)],
                    acc.at[pl.ds(s * _SLICE, _SLICE)])
    pltpu.sync_copy(u_hbm.at[pl.ds(s * _SLICE, _SLICE)],
                    u_sp.at[pl.ds(s * _SLICE, _SLICE)])
    plsc.subcore_barrier()

    @pl.loop(0, _J // _SEG)
    def _(g):
        pltpu.sync_copy(row_hbm.at[w, pl.ds(g * _SEG, _SEG)], row_v)
        pltpu.sync_copy(col_hbm.at[w, pl.ds(g * _SEG, _SEG)], col_v)

        @pl.loop(0, _SEG, step=_NB)
        def _(j):
            cps = [pltpu.async_copy(u_sp.at[row_v.at[j + b]],
                                    bufs[b], gsem[b])
                   for b in range(_NB)]
            for b in range(_NB):
                cps[b].wait()
                pltpu.sync_copy(bufs[b], acc.at[col_v.at[j + b]], add=True)

    plsc.subcore_barrier()
    pltpu.sync_copy(acc.at[pl.ds(s * _SLICE, _SLICE)],
                    part_hbm.at[c, pl.ds(s * _SLICE, _SLICE)])


def _mlp_body(x_ref, w1_ref, b1_ref, w2_ref, b2_ref, h_ref):
    h1 = jnp.maximum(x_ref[...] @ w1_ref[...] + b1_ref[...], 0.0)
    h_ref[...] = h1 @ w2_ref[...] + b2_ref[...]


def _mlp(xp, W1, b1, W2, b2):
    B = 1024
    g = _NP // B
    fullspec = lambda shape: pl.BlockSpec(shape, lambda i: (0, 0))
    rowspec = lambda shape: pl.BlockSpec(shape, lambda i: (i, 0))
    return pl.pallas_call(
        _mlp_body,
        grid=(g,),
        in_specs=[rowspec((B, _IN_C)), fullspec((_IN_C, _HID_C)),
                  fullspec((1, _HID_C)), fullspec((_HID_C, _OUT_C)),
                  fullspec((1, _OUT_C))],
        out_specs=rowspec((B, _OUT_C)),
        out_shape=jax.ShapeDtypeStruct((_NP, _OUT_C), jnp.float32),
    )(xp, W1, b1, W2, b2)


def _scale_body(h_ref, d0_ref, d1_ref, u0_ref, dis_ref):
    h = h_ref[...]
    deg = d0_ref[:, 0:1] + d1_ref[:, 0:1]
    dis = jnp.where(deg > 0.0, lax.rsqrt(deg), 0.0)
    u0_ref[...] = dis * h
    dis_ref[...] = jnp.broadcast_to(dis, h.shape)


def _scale(h, d0, d1):
    B = 1024
    g = _NP // B
    rowspec = lambda shape: pl.BlockSpec(shape, lambda i: (i, 0))
    return pl.pallas_call(
        _scale_body,
        grid=(g,),
        in_specs=[rowspec((B, _OUT_C)), rowspec((B, 16)), rowspec((B, 16))],
        out_specs=[rowspec((B, _OUT_C))] * 2,
        out_shape=[jax.ShapeDtypeStruct((_NP, _OUT_C), jnp.float32)] * 2,
    )(h, d0, d1)


def _comb_body(p0_ref, p1_ref, dis_ref, u0_ref, out_ref):
    dis = dis_ref[...]
    agg = p0_ref[...] + p1_ref[...]
    out_ref[...] = (1.0 - _ALPHA) * dis * dis * agg + _ALPHA * u0_ref[...]


def _comb(p0, p1, dis, u0):
    B = 1024
    g = _NP // B
    spec = pl.BlockSpec((B, _OUT_C), lambda i: (i, 0))
    return pl.pallas_call(
        _comb_body,
        grid=(g,),
        in_specs=[spec] * 4,
        out_specs=spec,
        out_shape=jax.ShapeDtypeStruct((_NP, _OUT_C), jnp.float32),
    )(p0, p1, dis, u0)


def _fin_body(p0_ref, p1_ref, dis_ref, h_ref, out_ref):
    t = ((1.0 - _ALPHA) * dis_ref[...] * (p0_ref[...] + p1_ref[...])
         + _ALPHA * h_ref[...])
    m = jnp.max(t, axis=1, keepdims=True)
    e = jnp.exp(t - m)
    out_ref[...] = (t - m) - jnp.log(jnp.sum(e, axis=1, keepdims=True))


def _fin(p0, p1, dis, h):
    B = 1000
    g = _N // B
    spec = pl.BlockSpec((B, _OUT_C), lambda i: (i, 0))
    return pl.pallas_call(
        _fin_body,
        grid=(g,),
        in_specs=[spec] * 4,
        out_specs=spec,
        out_shape=jax.ShapeDtypeStruct((_N, _OUT_C), jnp.float32),
    )(p0, p1, dis, h)


def kernel(x, edge_index, W1, b1, W2, b2):
    row = edge_index[0].astype(jnp.int32)
    col = edge_index[1].astype(jnp.int32)
    loop = jnp.arange(_N, dtype=jnp.int32)
    pad = _EPAD - _ET
    row = jnp.concatenate([row, loop, jnp.zeros((pad,), jnp.int32)])
    col = jnp.concatenate(
        [col, loop, _N + (jnp.arange(pad, dtype=jnp.int32) % (_NP - _N))])
    rowt = row.reshape(_J, _TILES, _CH).transpose(1, 0, 2)
    colt = col.reshape(_J, _TILES, _CH).transpose(1, 0, 2)

    zer = jnp.zeros((_NP, _OUT_C), jnp.float32)
    zer16 = jnp.zeros((1, _NP, 16), jnp.float32)
    ones16 = jnp.ones((_CH, 16), jnp.float32)

    dpart = _deg_kernel(colt, zer16, ones16)
    xp = jnp.concatenate([x, jnp.zeros((_NP - _N, _IN_C), jnp.float32)])
    h = _mlp(xp, W1, b1.reshape(1, -1), W2, b2.reshape(1, -1))
    u0, dis = _scale(h, dpart[0], dpart[1])

    u = u0
    for _ in range(_K_HOPS - 1):
        p = _hop_kernel(u, rowt, colt, zer)
        u = _comb(p[0], p[1], dis, u0)
    p = _hop_kernel(u, rowt, colt, zer)
    return _fin(p[0][:_N], p[1][:_N], dis[:_N], h[:_N])
